# baseline (speedup 1.0000x reference)
import functools

import jax
import jax.numpy as jnp
from jax import lax
from jax.experimental import pallas as pl
from jax.experimental.pallas import tpu as pltpu
from jax.experimental.pallas import tpu_sc as plsc

_B, _N, _H, _K = 4, 2048, 768, 256
_BIG = 2 ** 30


def _sc_gather(aa2, ao2, oo2, oa2, ija):
    info = plsc.get_sparse_core_info()
    nc = info.num_cores
    mesh = plsc.VectorSubcoreMesh(core_axis_name="c", subcore_axis_name="s")

    @functools.partial(
        pl.kernel,
        mesh=mesh,
        out_type=jax.ShapeDtypeStruct((_B * 6 * _K, _H), jnp.float32),
        scratch_types=[
            pltpu.VMEM((6, 32), jnp.int32),
            pltpu.VMEM((32, _H), jnp.float32),
            pltpu.VMEM((32, _H), jnp.float32),
            pltpu.SemaphoreType.DMA,
            pltpu.SemaphoreType.DMA,
        ],
    )
    def gk(aa_h, ao_h, oo_h, oa_h, ija_h, out_h, idx_s, buf_a, buf_b, semg, semw):
        w = lax.axis_index("s") * nc + lax.axis_index("c")
        b = w // 8
        r = w % 8
        slot = r // 4
        c4 = r % 4
        base = b * _N
        pltpu.sync_copy(ija_h.at[slot, b, pl.ds(c4 * 64, 32)], idx_s.at[0])
        pltpu.sync_copy(ija_h.at[slot, b, pl.ds(c4 * 64 + 32, 32)], idx_s.at[1])
        pltpu.sync_copy(ija_h.at[2 + slot, b, pl.ds(c4 * 64, 32)], idx_s.at[2])
        pltpu.sync_copy(ija_h.at[2 + slot, b, pl.ds(c4 * 64 + 32, 32)], idx_s.at[3])
        pltpu.sync_copy(ija_h.at[1, b, pl.ds(r * 32, 32)], idx_s.at[4])
        pltpu.sync_copy(ija_h.at[2, b, pl.ds(r * 32, 32)], idx_s.at[5])
        for k in range(6):
            for i in range(2):
                idx_s[k, pl.ds(i * 16, 16)] = idx_s[k, pl.ds(i * 16, 16)] + base
        tables = [aa_h, aa_h, oo_h, oo_h, ao_h, oa_h]
        bases = [
            (b * 6 + slot) * 256 + c4 * 64,
            (b * 6 + slot) * 256 + c4 * 64 + 32,
            (b * 6 + 3 + slot) * 256 + c4 * 64,
            (b * 6 + 3 + slot) * 256 + c4 * 64 + 32,
            (b * 6 + 2) * 256 + r * 32,
            (b * 6 + 5) * 256 + r * 32,
        ]
        bufs = [buf_a, buf_b]
        gather = pltpu.async_copy(tables[0].at[idx_s.at[0]], bufs[0], semg)
        writes = []
        for k in range(6):
            gather.wait()
            if k >= 1:
                writes[k - 1].wait()
            writes.append(pltpu.async_copy(
                bufs[k % 2], out_h.at[pl.ds(bases[k], 32)], semw))
            if k + 1 < 6:
                gather = pltpu.async_copy(
                    tables[k + 1].at[idx_s.at[k + 1]], bufs[(k + 1) % 2], semg)
        writes[5].wait()

    return gk(aa2, ao2, oo2, oa2, ija)


def _tc_body(g_ref, ia_ref, ja_ref, io_ref, jo_ref, lab_ref, wat_ref, wot_ref,
             bp_ref, ao0_ref, oo0_ref, fl_ref, pred_ref, loss_ref):
    b = pl.program_id(0)
    K, H, N = _K, _H, _N
    lower = lax.broadcasted_iota(jnp.int32, (K, K), 1) < \
        lax.broadcasted_iota(jnp.int32, (K, K), 0)
    iota_kn = lax.broadcasted_iota(jnp.int32, (K, N), 1)
    nt = (((1,), (1,)), ((), ()))

    def direction(Ha, Ho, Oo, Aa, asp_row, opi_row, wt_ref, b_col, o0):
        w_top_t = wt_ref[:, :H]
        w_bot_t = wt_ref[:, H:]
        asp_col = asp_row.reshape(K, 1)
        opi_col = opi_row.reshape(K, 1)
        st = lax.dot_general(Ho, Ha, nt, preferred_element_type=jnp.float32) * 0.01
        neqt = (opi_col != asp_row).astype(jnp.float32)
        ssct = st * neqt
        dup_o = jnp.any((opi_col == opi_row) & lower, axis=1, keepdims=True)
        colmask = jnp.where(dup_o, 0.0, 1.0)
        row_sum = jnp.sum(ssct * colmask, axis=0, keepdims=True)
        at = jnp.exp(ssct - row_sum) * neqt
        max_a = jnp.max(at, axis=0, keepdims=True)
        has = max_a > 0.0
        is_max = (at == max_a) & (neqt > 0.0) & has
        opi_b = jnp.broadcast_to(opi_col, (K, K))
        jstar = jnp.min(jnp.where(is_max, opi_b, _BIG), axis=0, keepdims=True)
        sel = (is_max & (opi_b == jstar)).astype(jnp.float32)
        cnt = jnp.sum(sel, axis=0, keepdims=True)
        hi = lax.Precision.HIGHEST
        proj_ot = lax.dot_general(w_bot_t, Oo, nt,
                                  preferred_element_type=jnp.float32)
        pselt = jnp.dot(proj_ot, sel, precision=hi,
                        preferred_element_type=jnp.float32) / \
            jnp.maximum(cnt, 1.0)
        proj0t = lax.dot_general(w_bot_t, o0, nt,
                                 preferred_element_type=jnp.float32)
        pselt = jnp.where(has, pselt, jnp.broadcast_to(proj0t, (3, K)))
        g_at = lax.dot_general(w_top_t, Aa, nt,
                               preferred_element_type=jnp.float32)
        lt = g_at + pselt + b_col
        dup_a = jnp.any((asp_col == asp_row) & lower, axis=1, keepdims=True)
        rowmask = jnp.where(dup_a, 0.0, 1.0)
        oh = (asp_col == iota_kn).astype(jnp.float32) * rowmask
        return jnp.dot(lt, oh, precision=hi, preferred_element_type=jnp.float32)

    fl1 = direction(g_ref[0, 0], g_ref[0, 1], g_ref[0, 2], g_ref[0, 0],
                    ia_ref[0], ja_ref[0], wat_ref, bp_ref[0:3, 0:1], ao0_ref[0])
    fl2 = direction(g_ref[0, 3], g_ref[0, 4], g_ref[0, 4], g_ref[0, 5],
                    io_ref[0], jo_ref[0], wot_ref, bp_ref[0:3, 1:2], oo0_ref[0])
    fl = 0.5 * (fl1 + fl2)
    valid = jnp.sum(jnp.abs(fl), axis=0, keepdims=True) > 0.0
    m = jnp.max(fl, axis=0, keepdims=True)
    ex = jnp.exp(fl - m)
    logp = fl - m - jnp.log(jnp.sum(ex, axis=0, keepdims=True))
    lab_row = lab_ref[0]
    nll = -jnp.where(lab_row == 0, logp[0:1, :],
                     jnp.where(lab_row == 1, logp[1:2, :], logp[2:3, :]))
    wlab = jnp.where(lab_row == 0, 1.0, jnp.where(lab_row == 1, 2.0, 4.0))
    wl = wlab * valid.astype(jnp.float32)
    loss_b = (jnp.sum(nll * wl) / jnp.maximum(jnp.sum(wl), 1e-6)).reshape(1, 1)
    f0, f1, f2 = fl[0:1, :], fl[1:2, :], fl[2:3, :]
    p01 = jnp.where(f1 > f0, 1, 0)
    pidx = jnp.where(f2 > jnp.maximum(f0, f1), 2, p01)
    pred = jnp.where(valid, pidx, -1)
    fl_ref[0] = fl
    pred_ref[0] = pred

    @pl.when(b == 0)
    def _():
        loss_ref[:, :] = jnp.zeros((1, 1), jnp.float32)

    loss_ref[:, :] += loss_b


_TC_GRID = (_B,)
_TC_IN_SPECS = [
    pl.BlockSpec((1, 6, _K, _H), lambda b: (b, 0, 0, 0)),
    pl.BlockSpec((1, 1, _K), lambda b: (b, 0, 0)),
    pl.BlockSpec((1, 1, _K), lambda b: (b, 0, 0)),
    pl.BlockSpec((1, 1, _K), lambda b: (b, 0, 0)),
    pl.BlockSpec((1, 1, _K), lambda b: (b, 0, 0)),
    pl.BlockSpec((1, 1, _N), lambda b: (b, 0, 0)),
    pl.BlockSpec((3, 2 * _H), lambda b: (0, 0)),
    pl.BlockSpec((3, 2 * _H), lambda b: (0, 0)),
    pl.BlockSpec((8, 128), lambda b: (0, 0)),
    pl.BlockSpec((1, 1, _H), lambda b: (b, 0, 0)),
    pl.BlockSpec((1, 1, _H), lambda b: (b, 0, 0)),
]
_TC_OUT_SPECS = [
    pl.BlockSpec((1, 3, _N), lambda b: (b, 0, 0)),
    pl.BlockSpec((1, 1, _N), lambda b: (b, 0, 0)),
    pl.BlockSpec((1, 1), lambda b: (0, 0)),
]
_TC_OUT_SHAPE = [
    jax.ShapeDtypeStruct((_B, 3, _N), jnp.float32),
    jax.ShapeDtypeStruct((_B, 1, _N), jnp.int32),
    jax.ShapeDtypeStruct((1, 1), jnp.float32),
]


def kernel(A2O_aspect_hidden_states, A2O_opinion_hidden_states,
           O2A_aspect_hidden_states, O2A_opinion_hidden_states,
           W_A2O, b_A2O, W_O2A, b_O2A,
           asp_idx_a2o, opi_idx_a2o, asp_idx_o2a, opi_idx_o2a,
           sentiment_labels):
    aa, ao = A2O_aspect_hidden_states, A2O_opinion_hidden_states
    oa, oo = O2A_aspect_hidden_states, O2A_opinion_hidden_states
    ija = jnp.stack([asp_idx_a2o, opi_idx_a2o,
                     asp_idx_o2a, opi_idx_o2a]).astype(jnp.int32)
    g = _sc_gather(aa.reshape(_B * _N, _H), ao.reshape(_B * _N, _H),
                   oo.reshape(_B * _N, _H), oa.reshape(_B * _N, _H),
                   ija).reshape(_B, 6, _K, _H)
    bpad = jnp.zeros((8, 128), jnp.float32).at[:3, 0].set(b_A2O).at[:3, 1].set(b_O2A)
    fl_t, pred, loss = pl.pallas_call(
        _tc_body,
        grid=_TC_GRID,
        in_specs=_TC_IN_SPECS,
        out_specs=_TC_OUT_SPECS,
        out_shape=_TC_OUT_SHAPE,
    )(g, ija[0].reshape(_B, 1, _K), ija[1].reshape(_B, 1, _K),
      ija[2].reshape(_B, 1, _K), ija[3].reshape(_B, 1, _K),
      sentiment_labels.astype(jnp.int32).reshape(_B, 1, _N),
      W_A2O.T, W_O2A.T, bpad, ao[:, 0:1, :], oo[:, 0:1, :])
    return jnp.swapaxes(fl_t, 1, 2), pred.reshape(_B, _N), loss[0, 0]


# speedup vs baseline: 1.0403x; 1.0403x over previous
import functools

import jax
import jax.numpy as jnp
from jax import lax
from jax.experimental import pallas as pl
from jax.experimental.pallas import tpu as pltpu
from jax.experimental.pallas import tpu_sc as plsc

_B, _N, _H, _K = 4, 2048, 768, 256
_BIG = 2 ** 30


def _sc_gather_dir(t64, t32, ija, i64, i32):
    info = plsc.get_sparse_core_info()
    nc = info.num_cores
    mesh = plsc.VectorSubcoreMesh(core_axis_name="c", subcore_axis_name="s")

    @functools.partial(
        pl.kernel,
        mesh=mesh,
        out_type=jax.ShapeDtypeStruct((_B * 3 * _K, _H), jnp.float32),
        scratch_types=[
            pltpu.VMEM((64,), jnp.int32),
            pltpu.VMEM((64, _H), jnp.float32),
            pltpu.VMEM((32,), jnp.int32),
            pltpu.VMEM((32, _H), jnp.float32),
            pltpu.SemaphoreType.DMA,
        ],
    )
    def gk(t64_h, t32_h, ija_h, out_h, idx64, rows64, idx32, rows32, sem):
        w = lax.axis_index("s") * nc + lax.axis_index("c")
        b = w // 8
        r = w % 8
        slot = r // 4
        c4 = r % 4
        base = b * _N
        pltpu.sync_copy(ija_h.at[i64 + slot, b, pl.ds(c4 * 64, 64)], idx64)
        for i in range(4):
            idx64[pl.ds(i * 16, 16)] = idx64[pl.ds(i * 16, 16)] + base
        pltpu.async_copy(t64_h.at[idx64], rows64, sem).wait()
        pltpu.sync_copy(rows64, out_h.at[pl.ds((b * 3 + slot) * 256 + c4 * 64, 64)])
        pltpu.sync_copy(ija_h.at[i32, b, pl.ds(r * 32, 32)], idx32)
        for i in range(2):
            idx32[pl.ds(i * 16, 16)] = idx32[pl.ds(i * 16, 16)] + base
        pltpu.async_copy(t32_h.at[idx32], rows32, sem).wait()
        pltpu.sync_copy(rows32, out_h.at[pl.ds((b * 3 + 2) * 256 + r * 32, 32)])

    return gk(t64, t32, ija)


def _direction(Ha, Ho, Oo, Aa, asp_row, opi_row, wt_ref, b_col, o0):
    K, H, N = _K, _H, _N
    nt = (((1,), (1,)), ((), ()))
    hi = lax.Precision.HIGHEST
    lower = lax.broadcasted_iota(jnp.int32, (K, K), 1) < \
        lax.broadcasted_iota(jnp.int32, (K, K), 0)
    w_top_t = wt_ref[:, :H]
    w_bot_t = wt_ref[:, H:]
    asp_col = asp_row.reshape(K, 1)
    opi_col = opi_row.reshape(K, 1)
    st = lax.dot_general(Ho, Ha, nt, preferred_element_type=jnp.float32) * 0.01
    neqt = (opi_col != asp_row).astype(jnp.float32)
    ssct = st * neqt
    dup_o = jnp.any((opi_col == opi_row) & lower, axis=1, keepdims=True)
    colmask = jnp.where(dup_o, 0.0, 1.0)
    row_sum = jnp.sum(ssct * colmask, axis=0, keepdims=True)
    at = jnp.exp(ssct - row_sum) * neqt
    max_a = jnp.max(at, axis=0, keepdims=True)
    has = max_a > 0.0
    is_max = (at == max_a) & (neqt > 0.0) & has
    opi_b = jnp.broadcast_to(opi_col, (K, K))
    jstar = jnp.min(jnp.where(is_max, opi_b, _BIG), axis=0, keepdims=True)
    sel = (is_max & (opi_b == jstar)).astype(jnp.float32)
    cnt = jnp.sum(sel, axis=0, keepdims=True)
    proj_ot = lax.dot_general(w_bot_t, Oo, nt, preferred_element_type=jnp.float32)
    pselt = jnp.dot(proj_ot, sel, precision=hi,
                    preferred_element_type=jnp.float32) / jnp.maximum(cnt, 1.0)
    proj0t = lax.dot_general(w_bot_t, o0, nt, preferred_element_type=jnp.float32)
    pselt = jnp.where(has, pselt, jnp.broadcast_to(proj0t, (3, K)))
    g_at = lax.dot_general(w_top_t, Aa, nt, preferred_element_type=jnp.float32)
    lt = g_at + pselt + b_col
    dup_a = jnp.any((asp_col == asp_row) & lower, axis=1, keepdims=True)
    rowmask = jnp.where(dup_a, 0.0, 1.0)
    iota_kn = lax.broadcasted_iota(jnp.int32, (K, N), 1)
    oh = (asp_col == iota_kn).astype(jnp.float32) * rowmask
    return jnp.dot(lt, oh, precision=hi, preferred_element_type=jnp.float32)


def _tca_body(g_ref, ia_ref, ja_ref, wat_ref, bp_ref, ao0_ref, fl1_ref):
    fl1_ref[0] = _direction(g_ref[0, 0], g_ref[0, 1], g_ref[0, 2], g_ref[0, 0],
                            ia_ref[0], ja_ref[0], wat_ref,
                            bp_ref[0:3, 0:1], ao0_ref[0])


def _tcb_body(g_ref, io_ref, jo_ref, lab_ref, wot_ref, bp_ref, oo0_ref,
              fl1_ref, fl_ref, pred_ref, loss_ref):
    b = pl.program_id(0)
    N = _N
    fl2 = _direction(g_ref[0, 0], g_ref[0, 1], g_ref[0, 1], g_ref[0, 2],
                     io_ref[0], jo_ref[0], wot_ref,
                     bp_ref[0:3, 1:2], oo0_ref[0])
    fl = 0.5 * (fl1_ref[0] + fl2)
    valid = jnp.sum(jnp.abs(fl), axis=0, keepdims=True) > 0.0
    m = jnp.max(fl, axis=0, keepdims=True)
    ex = jnp.exp(fl - m)
    logp = fl - m - jnp.log(jnp.sum(ex, axis=0, keepdims=True))
    lab_row = lab_ref[0]
    nll = -jnp.where(lab_row == 0, logp[0:1, :],
                     jnp.where(lab_row == 1, logp[1:2, :], logp[2:3, :]))
    wlab = jnp.where(lab_row == 0, 1.0, jnp.where(lab_row == 1, 2.0, 4.0))
    wl = wlab * valid.astype(jnp.float32)
    loss_b = (jnp.sum(nll * wl) / jnp.maximum(jnp.sum(wl), 1e-6)).reshape(1, 1)
    f0, f1, f2 = fl[0:1, :], fl[1:2, :], fl[2:3, :]
    p01 = jnp.where(f1 > f0, 1, 0)
    pidx = jnp.where(f2 > jnp.maximum(f0, f1), 2, p01)
    pred = jnp.where(valid, pidx, -1)
    fl_ref[0] = fl
    pred_ref[0] = pred

    @pl.when(b == 0)
    def _():
        loss_ref[:, :] = jnp.zeros((1, 1), jnp.float32)

    loss_ref[:, :] += loss_b


_SPEC_G = pl.BlockSpec((1, 3, _K, _H), lambda b: (b, 0, 0, 0))
_SPEC_K = pl.BlockSpec((1, 1, _K), lambda b: (b, 0, 0))
_SPEC_N = pl.BlockSpec((1, 1, _N), lambda b: (b, 0, 0))
_SPEC_W = pl.BlockSpec((3, 2 * _H), lambda b: (0, 0))
_SPEC_BP = pl.BlockSpec((8, 128), lambda b: (0, 0))
_SPEC_H1 = pl.BlockSpec((1, 1, _H), lambda b: (b, 0, 0))
_SPEC_FL = pl.BlockSpec((1, 3, _N), lambda b: (b, 0, 0))

_TCA_IN_SPECS = [_SPEC_G, _SPEC_K, _SPEC_K, _SPEC_W, _SPEC_BP, _SPEC_H1]
_TCA_OUT_SPECS = _SPEC_FL
_TCA_OUT_SHAPE = jax.ShapeDtypeStruct((_B, 3, _N), jnp.float32)

_TCB_IN_SPECS = [_SPEC_G, _SPEC_K, _SPEC_K, _SPEC_N, _SPEC_W, _SPEC_BP,
                 _SPEC_H1, _SPEC_FL]
_TCB_OUT_SPECS = [
    _SPEC_FL,
    _SPEC_N,
    pl.BlockSpec((1, 1), lambda b: (0, 0)),
]
_TCB_OUT_SHAPE = [
    jax.ShapeDtypeStruct((_B, 3, _N), jnp.float32),
    jax.ShapeDtypeStruct((_B, 1, _N), jnp.int32),
    jax.ShapeDtypeStruct((1, 1), jnp.float32),
]


def kernel(A2O_aspect_hidden_states, A2O_opinion_hidden_states,
           O2A_aspect_hidden_states, O2A_opinion_hidden_states,
           W_A2O, b_A2O, W_O2A, b_O2A,
           asp_idx_a2o, opi_idx_a2o, asp_idx_o2a, opi_idx_o2a,
           sentiment_labels):
    aa, ao = A2O_aspect_hidden_states, A2O_opinion_hidden_states
    oa, oo = O2A_aspect_hidden_states, O2A_opinion_hidden_states
    ija = jnp.stack([asp_idx_a2o, opi_idx_a2o,
                     asp_idx_o2a, opi_idx_o2a]).astype(jnp.int32)
    ga = _sc_gather_dir(aa.reshape(_B * _N, _H), ao.reshape(_B * _N, _H),
                        ija, 0, 1).reshape(_B, 3, _K, _H)
    gb = _sc_gather_dir(oo.reshape(_B * _N, _H), oa.reshape(_B * _N, _H),
                        ija, 2, 2).reshape(_B, 3, _K, _H)
    bpad = jnp.zeros((8, 128), jnp.float32).at[:3, 0].set(b_A2O).at[:3, 1].set(b_O2A)
    fl1 = pl.pallas_call(
        _tca_body,
        grid=(_B,),
        in_specs=_TCA_IN_SPECS,
        out_specs=_TCA_OUT_SPECS,
        out_shape=_TCA_OUT_SHAPE,
    )(ga, ija[0].reshape(_B, 1, _K), ija[1].reshape(_B, 1, _K),
      W_A2O.T, bpad, ao[:, 0:1, :])
    fl_t, pred, loss = pl.pallas_call(
        _tcb_body,
        grid=(_B,),
        in_specs=_TCB_IN_SPECS,
        out_specs=_TCB_OUT_SPECS,
        out_shape=_TCB_OUT_SHAPE,
    )(gb, ija[2].reshape(_B, 1, _K), ija[3].reshape(_B, 1, _K),
      sentiment_labels.astype(jnp.int32).reshape(_B, 1, _N),
      W_O2A.T, bpad, oo[:, 0:1, :], fl1)
    return jnp.swapaxes(fl_t, 1, 2), pred.reshape(_B, _N), loss[0, 0]
